# Initial kernel scaffold; baseline (speedup 1.0000x reference)
import jax
import jax.numpy as jnp
from jax.experimental import pallas as pl


def kernel(x, edge_index, W1, b1, as1, ad1, g1, be1, W2, b2, as2, ad2, g2, be2, W3, b3, as3, ad3, g3, be3, Wl, bl):
    raise NotImplementedError("write your pallas kernel here")


# speedup vs baseline: 75.7603x; 75.7603x over previous
import functools

import jax
import jax.numpy as jnp
from jax import lax
from jax.experimental import pallas as pl
from jax.experimental.pallas import tpu as pltpu
from jax.experimental.pallas import tpu_sc as plsc

N = 10000
E = 320000
D_IN = 128
HID = 64
HEADS = 8
DH = 8
OUT = 3

ROW = 80
CH = 80
NTILES = 32
EPT = E // NTILES
NCHUNK = EPT // CH
ROWS_PER_TILE = 624
ROWS_REM = N - 16 * ROWS_PER_TILE

BN = 400
GRID = N // BN


_sc_mesh = plsc.VectorSubcoreMesh(core_axis_name="c", subcore_axis_name="s")


@functools.partial(
    pl.kernel,
    out_type=jax.ShapeDtypeStruct((2, N, ROW), jnp.float32),
    mesh=_sc_mesh,
    compiler_params=pltpu.CompilerParams(needs_layout_passes=False,
                                         use_tc_tiling_on_sc=False),
    scratch_types=[
        pltpu.VMEM((CH,), jnp.int32),
        pltpu.VMEM((CH,), jnp.int32),
        pltpu.VMEM((CH, ROW), jnp.float32),
        pltpu.VMEM((CH, 8), jnp.float32),
        pltpu.VMEM((CH, ROW), jnp.float32),
        pltpu.VMEM((16,), jnp.float32),
        pltpu.VMEM_SHARED((N, ROW), jnp.float32),
        pltpu.SemaphoreType.DMA,
        pltpu.SemaphoreType.DMA,
    ],
)
def _sc_edge(src_hbm, dst_hbm, stab_hbm, ald_hbm, zeros_hbm, out_hbm,
             sidx, didx, rows, aldr, msg, exb, acc, sem_g, sem_a):
    cid = lax.axis_index("c")
    sid = lax.axis_index("s")
    wid = sid * 2 + cid

    zbase = sid * ROWS_PER_TILE
    pltpu.sync_copy(zeros_hbm.at[pl.ds(zbase, ROWS_PER_TILE)],
                    acc.at[pl.ds(zbase, ROWS_PER_TILE)])

    @pl.when(sid == 0)
    def _zero_rem():
        pltpu.sync_copy(zeros_hbm.at[pl.ds(16 * ROWS_PER_TILE, ROWS_REM)],
                        acc.at[pl.ds(16 * ROWS_PER_TILE, ROWS_REM)])

    zv = jnp.zeros((16,), jnp.float32)

    def _zpad(i, carry):
        msg[i, pl.ds(64, 16)] = zv
        return carry

    lax.fori_loop(0, CH, _zpad, 0)
    plsc.subcore_barrier()

    lane = lax.iota(jnp.int32, 16)
    col8 = lane & 7
    half = lane >> 3

    ebase = wid * EPT

    def _chunk(c, carry):
        off = ebase + c * CH
        pltpu.sync_copy(src_hbm.at[pl.ds(off, CH)], sidx)
        pltpu.sync_copy(dst_hbm.at[pl.ds(off, CH)], didx)
        ga = pltpu.async_copy(stab_hbm.at[sidx], rows, sem_g)
        gb = pltpu.async_copy(ald_hbm.at[didx], aldr, sem_a)
        ga.wait()
        gb.wait()

        def _pair(p, inner):
            e = 2 * p
            rvec = e + half
            als2 = plsc.load_gather(rows, [rvec, col8])
            ald2 = plsc.load_gather(aldr, [rvec, col8])
            a = als2 + ald2
            a = jnp.maximum(a, 0.2 * a)
            ex2 = jnp.exp(a)
            exb[...] = ex2
            plsc.store_scatter(msg, [rvec, col8], ex2)
            for k in range(4):
                i0 = half + 2 * k
                i1 = i0 + 8
                c0 = plsc.load_gather(exb, [i0])
                c1 = plsc.load_gather(exb, [i1])
                h0 = rows[e, pl.ds(8 + 16 * k, 16)]
                h1 = rows[e + 1, pl.ds(8 + 16 * k, 16)]
                msg[e, pl.ds(8 + 16 * k, 16)] = h0 * c0
                msg[e + 1, pl.ds(8 + 16 * k, 16)] = h1 * c1
            return inner

        lax.fori_loop(0, CH // 2, _pair, 0)
        pltpu.sync_copy(msg, acc.at[didx], add=True)
        return carry

    lax.fori_loop(0, NCHUNK, _chunk, 0)
    plsc.subcore_barrier()
    pltpu.sync_copy(acc.at[pl.ds(zbase, ROWS_PER_TILE)],
                    out_hbm.at[cid, pl.ds(zbase, ROWS_PER_TILE)])

    @pl.when(sid == 0)
    def _out_rem():
        pltpu.sync_copy(acc.at[pl.ds(16 * ROWS_PER_TILE, ROWS_REM)],
                        out_hbm.at[cid, pl.ds(16 * ROWS_PER_TILE, ROWS_REM)])


def _tc_pre_body(x_ref, wp_ref, bp_ref, wa_ref, ba_ref, stab_ref, ald_ref):
    xb = x_ref[...]
    stab_ref[...] = (
        jnp.dot(xb, wp_ref[...], preferred_element_type=jnp.float32)
        + bp_ref[...])
    ald_ref[...] = (
        jnp.dot(xb, wa_ref[...], preferred_element_type=jnp.float32)
        + ba_ref[...])


def _tc_pre(x, wp, bp, wa, ba):
    d = x.shape[1]
    return pl.pallas_call(
        _tc_pre_body,
        grid=(GRID,),
        in_specs=[
            pl.BlockSpec((BN, d), lambda i: (i, 0)),
            pl.BlockSpec((d, ROW), lambda i: (0, 0)),
            pl.BlockSpec((1, ROW), lambda i: (0, 0)),
            pl.BlockSpec((d, 8), lambda i: (0, 0)),
            pl.BlockSpec((1, 8), lambda i: (0, 0)),
        ],
        out_specs=[
            pl.BlockSpec((BN, ROW), lambda i: (i, 0)),
            pl.BlockSpec((BN, 8), lambda i: (i, 0)),
        ],
        out_shape=[
            jax.ShapeDtypeStruct((N, ROW), jnp.float32),
            jax.ShapeDtypeStruct((N, 8), jnp.float32),
        ],
    )(x, wp, bp, wa, ba)


def _combine(a0, a1, sd64_ref, snum_ref, g_ref, be_ref):
    t = a0 + a1
    d64 = jnp.dot(t, sd64_ref[...], preferred_element_type=jnp.float32)
    num = jnp.dot(t, snum_ref[...], preferred_element_type=jnp.float32)
    r = jnp.maximum(num / (d64 + 1e-16), 0.0)
    mu = jnp.mean(r, axis=1, keepdims=True)
    var = jnp.mean((r - mu) ** 2, axis=1, keepdims=True)
    return (r - mu) * lax.rsqrt(var + 1e-5) * g_ref[...] + be_ref[...]


def _tc_mid_body(a0_ref, a1_ref, sd64_ref, snum_ref, g_ref, be_ref,
                 wp_ref, bp_ref, wa_ref, ba_ref, stab_ref, ald_ref):
    y = _combine(a0_ref[...], a1_ref[...], sd64_ref, snum_ref, g_ref, be_ref)
    stab_ref[...] = (
        jnp.dot(y, wp_ref[...], preferred_element_type=jnp.float32)
        + bp_ref[...])
    ald_ref[...] = (
        jnp.dot(y, wa_ref[...], preferred_element_type=jnp.float32)
        + ba_ref[...])


def _tc_mid(a0, a1, sd64, snum, g, be, wp, bp, wa, ba):
    return pl.pallas_call(
        _tc_mid_body,
        grid=(GRID,),
        in_specs=[
            pl.BlockSpec((BN, ROW), lambda i: (i, 0)),
            pl.BlockSpec((BN, ROW), lambda i: (i, 0)),
            pl.BlockSpec((ROW, HID), lambda i: (0, 0)),
            pl.BlockSpec((ROW, HID), lambda i: (0, 0)),
            pl.BlockSpec((1, HID), lambda i: (0, 0)),
            pl.BlockSpec((1, HID), lambda i: (0, 0)),
            pl.BlockSpec((HID, ROW), lambda i: (0, 0)),
            pl.BlockSpec((1, ROW), lambda i: (0, 0)),
            pl.BlockSpec((HID, 8), lambda i: (0, 0)),
            pl.BlockSpec((1, 8), lambda i: (0, 0)),
        ],
        out_specs=[
            pl.BlockSpec((BN, ROW), lambda i: (i, 0)),
            pl.BlockSpec((BN, 8), lambda i: (i, 0)),
        ],
        out_shape=[
            jax.ShapeDtypeStruct((N, ROW), jnp.float32),
            jax.ShapeDtypeStruct((N, 8), jnp.float32),
        ],
    )(a0, a1, sd64, snum, g, be, wp, bp, wa, ba)


def _tc_fin_body(a0_ref, a1_ref, sd64_ref, snum_ref, g_ref, be_ref,
                 wl_ref, bl_ref, out_ref):
    y = _combine(a0_ref[...], a1_ref[...], sd64_ref, snum_ref, g_ref, be_ref)
    logits = (jnp.dot(y, wl_ref[...], preferred_element_type=jnp.float32)
              + bl_ref[...])
    z = logits - jnp.max(logits, axis=1, keepdims=True)
    ez = jnp.exp(z)
    out_ref[...] = ez / jnp.sum(ez, axis=1, keepdims=True)


def _tc_fin(a0, a1, sd64, snum, g, be, wl, bl):
    return pl.pallas_call(
        _tc_fin_body,
        grid=(GRID,),
        in_specs=[
            pl.BlockSpec((BN, ROW), lambda i: (i, 0)),
            pl.BlockSpec((BN, ROW), lambda i: (i, 0)),
            pl.BlockSpec((ROW, HID), lambda i: (0, 0)),
            pl.BlockSpec((ROW, HID), lambda i: (0, 0)),
            pl.BlockSpec((1, HID), lambda i: (0, 0)),
            pl.BlockSpec((1, HID), lambda i: (0, 0)),
            pl.BlockSpec((HID, OUT), lambda i: (0, 0)),
            pl.BlockSpec((1, OUT), lambda i: (0, 0)),
        ],
        out_specs=pl.BlockSpec((BN, OUT), lambda i: (i, 0)),
        out_shape=jax.ShapeDtypeStruct((N, OUT), jnp.float32),
    )(a0, a1, sd64, snum, g, be, wl, bl)


def _head_mat(a):
    eye = jnp.eye(HEADS, dtype=jnp.float32)
    return (a[:, :, None] * eye[:, None, :]).reshape(HID, HEADS)


def _pack(w, b, a_s, a_d):
    asm = _head_mat(a_s)
    adm = _head_mat(a_d)
    d = w.shape[0]
    wp = jnp.concatenate(
        [w @ asm, w, jnp.zeros((d, ROW - HEADS - HID), jnp.float32)], axis=1)
    bp = jnp.concatenate(
        [b @ asm, b, jnp.zeros((ROW - HEADS - HID,), jnp.float32)])
    return wp, bp[None, :], w @ adm, (b @ adm)[None, :]


def kernel(x, edge_index, W1, b1, as1, ad1, g1, be1, W2, b2, as2, ad2,
           g2, be2, W3, b3, as3, ad3, g3, be3, Wl, bl):
    src = edge_index[0]
    dst = edge_index[1]
    zeros = jnp.zeros((N, ROW), jnp.float32)

    snum = jnp.eye(ROW, HID, k=-HEADS, dtype=jnp.float32)
    rep = jnp.kron(jnp.eye(HEADS, dtype=jnp.float32),
                   jnp.ones((1, DH), jnp.float32))
    sd64 = jnp.eye(ROW, HEADS, dtype=jnp.float32) @ rep

    wp1, bp1, wa1, ba1 = _pack(W1, b1, as1, ad1)
    wp2, bp2, wa2, ba2 = _pack(W2, b2, as2, ad2)
    wp3, bp3, wa3, ba3 = _pack(W3, b3, as3, ad3)

    stab, ald = _tc_pre(x, wp1, bp1, wa1, ba1)
    acc = _sc_edge(src, dst, stab, ald, zeros)
    stab, ald = _tc_mid(acc[0], acc[1], sd64, snum, g1[None, :], be1[None, :],
                        wp2, bp2, wa2, ba2)
    acc = _sc_edge(src, dst, stab, ald, zeros)
    stab, ald = _tc_mid(acc[0], acc[1], sd64, snum, g2[None, :], be2[None, :],
                        wp3, bp3, wa3, ba3)
    acc = _sc_edge(src, dst, stab, ald, zeros)
    return _tc_fin(acc[0], acc[1], sd64, snum, g3[None, :], be3[None, :],
                   Wl, bl[None, :])
